# baseline (device time: 29323 ns/iter reference)
import jax
import jax.numpy as jnp
from jax import lax
from jax.experimental import pallas as pl
from jax.experimental.pallas import tpu as pltpu

N_DEV = 4
T = 512
V_LOC = 8192
V_CHUNK = 1024
N_CHUNKS = V_LOC // V_CHUNK
DMA_WINDOW = 3


def kernel(x, W, labels):
    def body(x_ref, w_hbm, lab_ref, out_ref,
             cnt_ref, comm_ref, send_sems, recv_sems):
        my_pos = lax.axis_index("i")

        barrier_sem = pltpu.get_barrier_semaphore()
        for d in (1, 2, 3):
            pl.semaphore_signal(
                barrier_sem, inc=1,
                device_id=((my_pos + d) % N_DEV,),
                device_id_type=pl.DeviceIdType.MESH,
            )

        x_f = x_ref[:, :]
        lab = lab_ref[:].reshape(T, 1)
        col0 = lax.broadcasted_iota(jnp.int32, (T, V_CHUNK), 1)
        comm_ref[my_pos, 0, :] = jnp.zeros((T,), jnp.float32)
        comm_ref[my_pos, 1, :] = jnp.zeros((T,), jnp.float32)
        cnt_ref[0] = 0

        def chunk_body(w_blk):
            c = cnt_ref[0]
            logits = jnp.dot(
                x_f, w_blk[:, :], preferred_element_type=jnp.float32
            )
            s_c = jnp.sum(jnp.exp(logits), axis=1, dtype=jnp.float32)
            lab_shift = lab - (my_pos * V_LOC + c * V_CHUNK)
            lab_c = jnp.sum(
                jnp.where(col0 == lab_shift, logits, 0.0),
                axis=1, dtype=jnp.float32,
            )
            comm_ref[my_pos, 0, :] = comm_ref[my_pos, 0, :] + s_c
            comm_ref[my_pos, 1, :] = comm_ref[my_pos, 1, :] + lab_c
            cnt_ref[0] = c + 1

        pltpu.emit_pipeline(
            chunk_body,
            grid=(N_CHUNKS,),
            in_specs=[pl.BlockSpec((1024, V_CHUNK), lambda c: (0, c))],
        )(w_hbm)

        pl.semaphore_wait(barrier_sem, 3)

        rdmas = []
        for d in (1, 2, 3):
            rdma = pltpu.make_async_remote_copy(
                src_ref=comm_ref.at[my_pos],
                dst_ref=comm_ref.at[my_pos],
                send_sem=send_sems.at[d],
                recv_sem=recv_sems.at[d],
                device_id=((my_pos + d) % N_DEV,),
                device_id_type=pl.DeviceIdType.MESH,
            )
            rdma.start()
            rdmas.append(rdma)
        for rdma in rdmas:
            rdma.wait_recv()

        stats = comm_ref[:, :, :]
        s_g = jnp.sum(stats[:, 0, :], axis=0)
        lab_logit = jnp.sum(stats[:, 1, :], axis=0)
        out_ref[:] = jnp.log(s_g) - lab_logit

        for rdma in rdmas:
            rdma.wait_send()

    out = pl.pallas_call(
        body,
        out_shape=jax.ShapeDtypeStruct((T,), jnp.float32),
        in_specs=[
            pl.BlockSpec(memory_space=pltpu.VMEM),
            pl.BlockSpec(memory_space=pl.ANY),
            pl.BlockSpec(memory_space=pltpu.VMEM),
        ],
        out_specs=pl.BlockSpec(memory_space=pltpu.VMEM),
        scratch_shapes=[
            pltpu.SMEM((1,), jnp.int32),
            pltpu.VMEM((N_DEV, 2, T), jnp.float32),
            pltpu.SemaphoreType.DMA((N_DEV,)),
            pltpu.SemaphoreType.DMA((N_DEV,)),
        ],
        compiler_params=pltpu.CompilerParams(
            collective_id=0, vmem_limit_bytes=100 * 1024 * 1024
        ),
    )(x, W, labels)
    return out


# device time: 24277 ns/iter; 1.2079x vs baseline; 1.2079x over previous
import jax
import jax.numpy as jnp
from jax import lax
from jax.experimental import pallas as pl
from jax.experimental.pallas import tpu as pltpu

N_DEV = 4
T = 512
V_LOC = 8192
V_CHUNK = 1024
N_CHUNKS = V_LOC // V_CHUNK
DMA_WINDOW = 8


def kernel(x, W, labels):
    def body(x_ref, w_hbm, lab_ref, out_ref,
             w_vmem, dma_sems, comm_ref, send_sems, recv_sems):
        my_pos = lax.axis_index("i")

        barrier_sem = pltpu.get_barrier_semaphore()
        for d in (1, 2, 3):
            pl.semaphore_signal(
                barrier_sem, inc=1,
                device_id=((my_pos + d) % N_DEV,),
                device_id_type=pl.DeviceIdType.MESH,
            )

        def make_copy(c):
            return pltpu.make_async_copy(
                w_hbm.at[:, pl.ds(c * V_CHUNK, V_CHUNK)],
                w_vmem.at[:, pl.ds(c * V_CHUNK, V_CHUNK)],
                dma_sems.at[c],
            )

        copies = [make_copy(c) for c in range(N_CHUNKS)]
        for c in range(DMA_WINDOW):
            copies[c].start()

        x16 = x_ref[:, :].astype(jnp.bfloat16)
        lab = lab_ref[:].reshape(T, 1)
        col0 = lax.broadcasted_iota(jnp.int32, (T, V_CHUNK), 1)
        s_run = lab_run = None
        for c in range(N_CHUNKS):
            copies[c].wait()
            if c + DMA_WINDOW < N_CHUNKS:
                copies[c + DMA_WINDOW].start()
            logits = jnp.dot(
                x_ref[:, :],
                w_vmem[:, c * V_CHUNK:(c + 1) * V_CHUNK],
                preferred_element_type=jnp.float32,
            )
            s_c = jnp.sum(jnp.exp(logits), axis=1, dtype=jnp.float32)
            lab_shift = lab - (my_pos * V_LOC + c * V_CHUNK)
            lab_c = jnp.sum(
                jnp.where(col0 == lab_shift, logits, 0.0),
                axis=1, dtype=jnp.float32,
            )
            if c == 0:
                s_run, lab_run = s_c, lab_c
            else:
                s_run = s_run + s_c
                lab_run = lab_run + lab_c

        comm_ref[my_pos, 0, :] = s_run
        comm_ref[my_pos, 1, :] = lab_run

        pl.semaphore_wait(barrier_sem, 3)

        rdmas = []
        for d in (1, 2, 3):
            rdma = pltpu.make_async_remote_copy(
                src_ref=comm_ref.at[my_pos],
                dst_ref=comm_ref.at[my_pos],
                send_sem=send_sems.at[d],
                recv_sem=recv_sems.at[d],
                device_id=((my_pos + d) % N_DEV,),
                device_id_type=pl.DeviceIdType.MESH,
            )
            rdma.start()
            rdmas.append(rdma)
        for rdma in rdmas:
            rdma.wait_recv()

        stats = comm_ref[:, :, :]
        s_g = jnp.sum(stats[:, 0, :], axis=0)
        lab_logit = jnp.sum(stats[:, 1, :], axis=0)
        out_ref[:] = jnp.log(s_g) - lab_logit

        for rdma in rdmas:
            rdma.wait_send()

    out = pl.pallas_call(
        body,
        out_shape=jax.ShapeDtypeStruct((T,), jnp.float32),
        in_specs=[
            pl.BlockSpec(memory_space=pltpu.VMEM),
            pl.BlockSpec(memory_space=pl.ANY),
            pl.BlockSpec(memory_space=pltpu.VMEM),
        ],
        out_specs=pl.BlockSpec(memory_space=pltpu.VMEM),
        scratch_shapes=[
            pltpu.VMEM((1024, V_LOC), jnp.float32),
            pltpu.SemaphoreType.DMA((N_CHUNKS,)),
            pltpu.VMEM((N_DEV, 2, T), jnp.float32),
            pltpu.SemaphoreType.DMA((N_DEV,)),
            pltpu.SemaphoreType.DMA((N_DEV,)),
        ],
        compiler_params=pltpu.CompilerParams(
            collective_id=0, vmem_limit_bytes=100 * 1024 * 1024
        ),
    )(x, W, labels)
    return out
